# Initial kernel scaffold; baseline (speedup 1.0000x reference)
import jax
import jax.numpy as jnp
from jax.experimental import pallas as pl


def kernel(x, edge_index, batch, W_enc, b_enc, W1, b1, W2, b2, eps, gamma, beta, rmean, rvar, Wh1, bh1, Wh2, bh2):
    raise NotImplementedError("write your pallas kernel here")


# speedup vs baseline: 2.3446x; 2.3446x over previous
import functools

import jax
import jax.numpy as jnp
from jax import lax
from jax.experimental import pallas as pl
from jax.experimental.pallas import tpu as pltpu
from jax.experimental.pallas import tpu_sc as plsc

NN = 10000
EE = 160000
DD = 512
HH = 256
LL = 5
GG = 64

NP = 10240
EP = 163840
FLAT = 2 * NP
CH = 128
RB = 1000
NBLK = NN // RB
ROWS_PER_TILE = NP // 16
CHUNKS_PER_TILE = EP // 16 // CH

_f32 = jnp.float32


def _enc_body(x_ref, w_ref, b_ref, o_ref):
    h = jnp.dot(x_ref[...], w_ref[...], preferred_element_type=_f32) + b_ref[...]
    o_ref[0] = h[:, :128]
    o_ref[1] = h[:, 128:]


def _encode(x, W_enc, b_enc2):
    return pl.pallas_call(
        _enc_body,
        grid=(NBLK,),
        in_specs=[
            pl.BlockSpec((RB, DD), lambda i: (i, 0)),
            pl.BlockSpec((DD, HH), lambda i: (0, 0)),
            pl.BlockSpec((1, HH), lambda i: (0, 0)),
        ],
        out_specs=pl.BlockSpec((2, RB, 128), lambda i: (0, i, 0)),
        out_shape=jax.ShapeDtypeStruct((2, NP, 128), _f32),
    )(x, W_enc, b_enc2)


def _mlp_body(h_ref, a_ref, s_ref, w1_ref, b1_ref, w2_ref, b2_ref, o_ref):
    h = jnp.concatenate([h_ref[0], h_ref[1]], axis=1)
    agg = jnp.concatenate([a_ref[0], a_ref[1]], axis=1)
    z = s_ref[0, 0] * h + agg
    y = jnp.maximum(jnp.dot(z, w1_ref[...], preferred_element_type=_f32) + b1_ref[...], 0.0)
    o = jnp.maximum(jnp.dot(y, w2_ref[...], preferred_element_type=_f32) + b2_ref[...], 0.0)
    o_ref[0] = o[:, :128]
    o_ref[1] = o[:, 128:]


def _mlp(h_flat, agg_flat, smul, W1l, b1l, W2l, b2l):
    return pl.pallas_call(
        _mlp_body,
        grid=(NBLK,),
        in_specs=[
            pl.BlockSpec((2, RB, 128), lambda i: (0, i, 0)),
            pl.BlockSpec((2, RB, 128), lambda i: (0, i, 0)),
            pl.BlockSpec((1, 1), lambda i: (0, 0)),
            pl.BlockSpec((HH, 2 * HH), lambda i: (0, 0)),
            pl.BlockSpec((1, 2 * HH), lambda i: (0, 0)),
            pl.BlockSpec((2 * HH, HH), lambda i: (0, 0)),
            pl.BlockSpec((1, HH), lambda i: (0, 0)),
        ],
        out_specs=pl.BlockSpec((2, RB, 128), lambda i: (0, i, 0)),
        out_shape=jax.ShapeDtypeStruct((2, NP, 128), _f32),
    )(h_flat, agg_flat, smul, W1l, b1l, W2l, b2l)


def _pool_body(h0, h1, h2, h3, h4, h5, b_ref, wh1_ref, bh1_ref, wh2_ref,
               bh2_ref, o_ref, pacc, cacc):
    i = pl.program_id(0)

    @pl.when(i == 0)
    def _():
        pacc[...] = jnp.zeros_like(pacc)
        cacc[...] = jnp.zeros_like(cacc)

    b = b_ref[0, 0, :]
    gi = lax.broadcasted_iota(jnp.int32, (GG, RB), 0)
    oneh = (gi == b[None, :]).astype(_f32)
    cnt = jnp.sum(oneh, axis=1, keepdims=True)
    cacc[...] += jnp.broadcast_to(cnt, (GG, 128))
    for l, hr in enumerate([h0, h1, h2, h3, h4, h5]):
        hl = jnp.concatenate([hr[0], hr[1]], axis=1)
        pacc[l] += jnp.dot(oneh, hl, preferred_element_type=_f32)

    @pl.when(i == NBLK - 1)
    def _():
        d = jnp.maximum(cacc[...], 1.0)
        d2 = jnp.concatenate([d, d], axis=1)
        acc = jnp.zeros((GG, HH), _f32)
        for l in range(LL + 1):
            acc += jnp.dot(pacc[l] / d2, wh1_ref[l], preferred_element_type=_f32)
        y = jnp.maximum(acc + bh1_ref[...], 0.0)
        o_ref[...] = jnp.dot(y, wh2_ref[...], preferred_element_type=_f32) + bh2_ref[0, 0]


def _pool_head(hs, batch3, Wh1r, bh12, Wh2p, bh22):
    return pl.pallas_call(
        _pool_body,
        grid=(NBLK,),
        in_specs=[pl.BlockSpec((2, RB, 128), lambda i: (0, i, 0)) for _ in range(6)]
        + [
            pl.BlockSpec((1, 1, RB), lambda i: (i, 0, 0)),
            pl.BlockSpec((LL + 1, HH, HH), lambda i: (0, 0, 0)),
            pl.BlockSpec((1, HH), lambda i: (0, 0)),
            pl.BlockSpec((HH, 128), lambda i: (0, 0)),
            pl.BlockSpec((1, 1), lambda i: (0, 0)),
        ],
        out_specs=pl.BlockSpec((GG, 128), lambda i: (0, 0)),
        out_shape=jax.ShapeDtypeStruct((GG, 128), _f32),
        scratch_shapes=[
            pltpu.VMEM((LL + 1, GG, HH), _f32),
            pltpu.VMEM((GG, 128), _f32),
        ],
    )(*hs, batch3, Wh1r, bh12, Wh2p, bh22)


def _sc_segsum_body(h_hbm, src_hbm, dst_hbm, zero_hbm, out_hbm,
                    sidx, didx, rows, acc, sem):
    c = lax.axis_index("c")
    s = lax.axis_index("s")
    pltpu.sync_copy(zero_hbm.at[pl.ds(s * ROWS_PER_TILE, ROWS_PER_TILE)],
                    acc.at[pl.ds(s * ROWS_PER_TILE, ROWS_PER_TILE)])
    plsc.subcore_barrier()

    ebase = s * (EP // 16)
    off = c * NP

    def body(k, carry):
        base = ebase + k * CH
        pltpu.sync_copy(src_hbm.at[pl.ds(base, CH)], sidx)
        pltpu.sync_copy(dst_hbm.at[pl.ds(base, CH)], didx)
        for j in range(CH // 16):
            sl = pl.ds(j * 16, 16)
            sidx[sl] = sidx[sl] + off
        pltpu.async_copy(h_hbm.at[sidx], rows, sem).wait()
        pltpu.sync_copy(rows, acc.at[didx], add=True)
        return carry

    lax.fori_loop(0, CHUNKS_PER_TILE, body, 0)

    plsc.subcore_barrier()
    pltpu.sync_copy(acc.at[pl.ds(s * ROWS_PER_TILE, ROWS_PER_TILE)],
                    out_hbm.at[pl.ds(off + s * ROWS_PER_TILE, ROWS_PER_TILE)])


def _segsum(h_flat2, srcp, dstp, zeros_half):
    mesh = plsc.VectorSubcoreMesh(core_axis_name="c", subcore_axis_name="s")
    f = functools.partial(
        pl.kernel,
        mesh=mesh,
        out_type=jax.ShapeDtypeStruct((FLAT, 128), _f32),
        scratch_types=[
            pltpu.VMEM((CH,), jnp.int32),
            pltpu.VMEM((CH,), jnp.int32),
            pltpu.VMEM((CH, 128), _f32),
            pltpu.VMEM_SHARED((NP, 128), _f32),
            pltpu.SemaphoreType.DMA,
        ],
    )(_sc_segsum_body)
    return f(h_flat2, srcp, dstp, zeros_half)


def kernel(x, edge_index, batch, W_enc, b_enc, W1, b1, W2, b2, eps, gamma,
           beta, rmean, rvar, Wh1, bh1, Wh2, bh2):
    src = edge_index[0]
    dst = edge_index[1]
    pad = EP - EE
    srcp = jnp.concatenate([src, jnp.zeros((pad,), jnp.int32)])
    dstp = jnp.concatenate([dst, jnp.full((pad,), NN, jnp.int32)])
    zeros_half = jnp.zeros((NP, 128), _f32)

    scale = gamma / jnp.sqrt(rvar + 1e-5)
    W2f = W2 * scale[:, None, :]
    b2f = (b2 - rmean) * scale + beta

    h_flat = _encode(x, W_enc, b_enc.reshape(1, HH))
    hs = [h_flat]
    for l in range(LL):
        agg = _segsum(h_flat.reshape(FLAT, 128), srcp, dstp, zeros_half)
        h_flat = _mlp(h_flat, agg.reshape(2, NP, 128),
                      (1.0 + eps[l]).reshape(1, 1),
                      W1[l], b1[l].reshape(1, 2 * HH),
                      W2f[l], b2f[l].reshape(1, HH))
        hs.append(h_flat)

    batch3 = batch.reshape(NBLK, 1, RB)
    Wh1r = Wh1.reshape(LL + 1, HH, HH)
    Wh2p = jnp.pad(Wh2, ((0, 0), (0, 127)))
    out = _pool_head(hs, batch3, Wh1r, bh1.reshape(1, HH), Wh2p,
                     bh2.reshape(1, 1))
    return out[:, 0]


# speedup vs baseline: 3.0331x; 1.2936x over previous
import functools

import jax
import jax.numpy as jnp
from jax import lax
from jax.experimental import pallas as pl
from jax.experimental.pallas import tpu as pltpu
from jax.experimental.pallas import tpu_sc as plsc

NN = 10000
EE = 160000
DD = 512
HH = 256
LL = 5
GG = 64

NP = 10240
EP = 163840
FLAT = 2 * NP
CH = 80
RB = 1000
NBLK = NN // RB
ROWS_PER_TILE = NP // 16
CHUNKS_PER_TILE = EP // 16 // CH

_f32 = jnp.float32


def _enc_body(x_ref, w_ref, b_ref, o_ref):
    h = jnp.dot(x_ref[...], w_ref[...], preferred_element_type=_f32) + b_ref[...]
    o_ref[0] = h[:, :128]
    o_ref[1] = h[:, 128:]


def _encode(x, W_enc, b_enc2):
    return pl.pallas_call(
        _enc_body,
        grid=(NBLK,),
        in_specs=[
            pl.BlockSpec((RB, DD), lambda i: (i, 0)),
            pl.BlockSpec((DD, HH), lambda i: (0, 0)),
            pl.BlockSpec((1, HH), lambda i: (0, 0)),
        ],
        out_specs=pl.BlockSpec((2, RB, 128), lambda i: (0, i, 0)),
        out_shape=jax.ShapeDtypeStruct((2, NP, 128), _f32),
    )(x, W_enc, b_enc2)


def _mlp_body(h_ref, a_ref, s_ref, w1_ref, b1_ref, w2_ref, b2_ref, o_ref):
    h = jnp.concatenate([h_ref[0], h_ref[1]], axis=1)
    agg = jnp.concatenate([a_ref[0], a_ref[1]], axis=1)
    z = s_ref[0, 0] * h + agg
    y = jnp.maximum(jnp.dot(z, w1_ref[...], preferred_element_type=_f32) + b1_ref[...], 0.0)
    o = jnp.maximum(jnp.dot(y, w2_ref[...], preferred_element_type=_f32) + b2_ref[...], 0.0)
    o_ref[0] = o[:, :128]
    o_ref[1] = o[:, 128:]


def _mlp(h_flat, agg_flat, smul, W1l, b1l, W2l, b2l):
    return pl.pallas_call(
        _mlp_body,
        grid=(NBLK,),
        in_specs=[
            pl.BlockSpec((2, RB, 128), lambda i: (0, i, 0)),
            pl.BlockSpec((2, RB, 128), lambda i: (0, i, 0)),
            pl.BlockSpec((1, 1), lambda i: (0, 0)),
            pl.BlockSpec((HH, 2 * HH), lambda i: (0, 0)),
            pl.BlockSpec((1, 2 * HH), lambda i: (0, 0)),
            pl.BlockSpec((2 * HH, HH), lambda i: (0, 0)),
            pl.BlockSpec((1, HH), lambda i: (0, 0)),
        ],
        out_specs=pl.BlockSpec((2, RB, 128), lambda i: (0, i, 0)),
        out_shape=jax.ShapeDtypeStruct((2, NP, 128), _f32),
    )(h_flat, agg_flat, smul, W1l, b1l, W2l, b2l)


def _pool_body(h0, h1, h2, h3, h4, h5, b_ref, wh1_ref, bh1_ref, wh2_ref,
               bh2_ref, o_ref, pacc, cacc):
    i = pl.program_id(0)

    @pl.when(i == 0)
    def _():
        pacc[...] = jnp.zeros_like(pacc)
        cacc[...] = jnp.zeros_like(cacc)

    b = b_ref[0, 0, :]
    gi = lax.broadcasted_iota(jnp.int32, (GG, RB), 0)
    oneh = (gi == b[None, :]).astype(_f32)
    cnt = jnp.sum(oneh, axis=1, keepdims=True)
    cacc[...] += jnp.broadcast_to(cnt, (GG, 128))
    for l, hr in enumerate([h0, h1, h2, h3, h4, h5]):
        hl = jnp.concatenate([hr[0], hr[1]], axis=1)
        pacc[l] += jnp.dot(oneh, hl, preferred_element_type=_f32)

    @pl.when(i == NBLK - 1)
    def _():
        d = jnp.maximum(cacc[...], 1.0)
        d2 = jnp.concatenate([d, d], axis=1)
        acc = jnp.zeros((GG, HH), _f32)
        for l in range(LL + 1):
            acc += jnp.dot(pacc[l] / d2, wh1_ref[l], preferred_element_type=_f32)
        y = jnp.maximum(acc + bh1_ref[...], 0.0)
        o_ref[...] = jnp.dot(y, wh2_ref[...], preferred_element_type=_f32) + bh2_ref[0, 0]


def _pool_head(hs, batch3, Wh1r, bh12, Wh2p, bh22):
    return pl.pallas_call(
        _pool_body,
        grid=(NBLK,),
        in_specs=[pl.BlockSpec((2, RB, 128), lambda i: (0, i, 0)) for _ in range(6)]
        + [
            pl.BlockSpec((1, 1, RB), lambda i: (i, 0, 0)),
            pl.BlockSpec((LL + 1, HH, HH), lambda i: (0, 0, 0)),
            pl.BlockSpec((1, HH), lambda i: (0, 0)),
            pl.BlockSpec((HH, 128), lambda i: (0, 0)),
            pl.BlockSpec((1, 1), lambda i: (0, 0)),
        ],
        out_specs=pl.BlockSpec((GG, 128), lambda i: (0, 0)),
        out_shape=jax.ShapeDtypeStruct((GG, 128), _f32),
        scratch_shapes=[
            pltpu.VMEM((LL + 1, GG, HH), _f32),
            pltpu.VMEM((GG, 128), _f32),
        ],
    )(*hs, batch3, Wh1r, bh12, Wh2p, bh22)


NSLOT = 2
NPHASE = 2
CPP = CHUNKS_PER_TILE // NPHASE
NGRP = CPP // NSLOT


def _sc_segsum_body(h_hbm, src4_hbm, dst4_hbm, zero_hbm, out_hbm, *scr):
    sidx_all, didx_all = scr[0], scr[1]
    rows = list(scr[2:2 + NSLOT])
    acc = scr[2 + NSLOT]
    gsem = list(scr[3 + NSLOT:3 + 2 * NSLOT])
    ssem = list(scr[3 + 2 * NSLOT:3 + 3 * NSLOT])
    c = lax.axis_index("c")
    s = lax.axis_index("s")
    pltpu.sync_copy(zero_hbm.at[pl.ds(s * ROWS_PER_TILE, ROWS_PER_TILE)],
                    acc.at[pl.ds(s * ROWS_PER_TILE, ROWS_PER_TILE)])
    plsc.subcore_barrier()

    for p in range(NPHASE):
        pltpu.sync_copy(src4_hbm.at[c, s, p], sidx_all)
        pltpu.sync_copy(dst4_hbm.at[s, p], didx_all)

        for j in range(NSLOT):
            pltpu.async_copy(h_hbm.at[sidx_all.at[j]], rows[j], gsem[j])

        def body(i, carry):
            for j in range(NSLOT):
                k = i * NSLOT + j
                pltpu.make_async_copy(h_hbm.at[sidx_all.at[k]], rows[j], gsem[j]).wait()
                pltpu.async_copy(rows[j], acc.at[didx_all.at[k]], ssem[j], add=True)
            for j in range(NSLOT):
                k = i * NSLOT + j
                pltpu.make_async_copy(rows[j], acc.at[didx_all.at[k]], ssem[j]).wait()

                @pl.when(i < NGRP - 1)
                def _():
                    pltpu.async_copy(h_hbm.at[sidx_all.at[k + NSLOT]], rows[j], gsem[j])
            return carry

        lax.fori_loop(0, NGRP, body, 0)

    plsc.subcore_barrier()
    pltpu.sync_copy(acc.at[pl.ds(s * ROWS_PER_TILE, ROWS_PER_TILE)],
                    out_hbm.at[pl.ds(c * NP + s * ROWS_PER_TILE, ROWS_PER_TILE)])


def _segsum(h_flat2, src4, dst4, zeros_half):
    mesh = plsc.VectorSubcoreMesh(core_axis_name="c", subcore_axis_name="s")
    f = functools.partial(
        pl.kernel,
        mesh=mesh,
        out_type=jax.ShapeDtypeStruct((FLAT, 128), _f32),
        scratch_types=[
            pltpu.VMEM((CPP, CH), jnp.int32),
            pltpu.VMEM((CPP, CH), jnp.int32),
        ]
        + [pltpu.VMEM((CH, 128), _f32) for _ in range(NSLOT)]
        + [pltpu.VMEM_SHARED((NP, 128), _f32)]
        + [pltpu.SemaphoreType.DMA for _ in range(2 * NSLOT)],
    )(_sc_segsum_body)
    return f(h_flat2, src4, dst4, zeros_half)


def kernel(x, edge_index, batch, W_enc, b_enc, W1, b1, W2, b2, eps, gamma,
           beta, rmean, rvar, Wh1, bh1, Wh2, bh2):
    src = edge_index[0]
    dst = edge_index[1]
    pad = EP - EE
    srcp = jnp.concatenate([src, jnp.zeros((pad,), jnp.int32)])
    dstp = jnp.concatenate([dst, jnp.full((pad,), NN, jnp.int32)])
    src4 = jnp.stack([srcp, srcp + NP]).reshape(2, 16, NPHASE, CPP, CH)
    dst4 = dstp.reshape(16, NPHASE, CPP, CH)
    zeros_half = jnp.zeros((NP, 128), _f32)

    scale = gamma / jnp.sqrt(rvar + 1e-5)
    W2f = W2 * scale[:, None, :]
    b2f = (b2 - rmean) * scale + beta

    h_flat = _encode(x, W_enc, b_enc.reshape(1, HH))
    hs = [h_flat]
    for l in range(LL):
        agg = _segsum(h_flat.reshape(FLAT, 128), src4, dst4, zeros_half)
        h_flat = _mlp(h_flat, agg.reshape(2, NP, 128),
                      (1.0 + eps[l]).reshape(1, 1),
                      W1[l], b1[l].reshape(1, 2 * HH),
                      W2f[l], b2f[l].reshape(1, HH))
        hs.append(h_flat)

    batch3 = batch.reshape(NBLK, 1, RB)
    Wh1r = Wh1.reshape(LL + 1, HH, HH)
    Wh2p = jnp.pad(Wh2, ((0, 0), (0, 127)))
    out = _pool_head(hs, batch3, Wh1r, bh1.reshape(1, HH), Wh2p,
                     bh2.reshape(1, 1))
    return out[:, 0]


# speedup vs baseline: 3.2457x; 1.0701x over previous
import functools

import jax
import jax.numpy as jnp
from jax import lax
from jax.experimental import pallas as pl
from jax.experimental.pallas import tpu as pltpu
from jax.experimental.pallas import tpu_sc as plsc

NN = 10000
EE = 160000
DD = 512
HH = 256
LL = 5
GG = 64

NP = 10240
EP = 163840
FLAT = 2 * NP
CH = 80
RB = 1000
NBLK = NN // RB
ROWS_PER_TILE = NP // 16
CHUNKS_PER_TILE = EP // 16 // CH

_f32 = jnp.float32


def _enc_body(x_ref, w_ref, b_ref, o_ref):
    h = jnp.dot(x_ref[...], w_ref[...], preferred_element_type=_f32) + b_ref[...]
    o_ref[0] = h[:, :128]
    o_ref[1] = h[:, 128:]


def _encode(x, W_enc, b_enc2):
    return pl.pallas_call(
        _enc_body,
        grid=(NBLK,),
        in_specs=[
            pl.BlockSpec((RB, DD), lambda i: (i, 0)),
            pl.BlockSpec((DD, HH), lambda i: (0, 0)),
            pl.BlockSpec((1, HH), lambda i: (0, 0)),
        ],
        out_specs=pl.BlockSpec((2, RB, 128), lambda i: (0, i, 0)),
        out_shape=jax.ShapeDtypeStruct((2, NP, 128), _f32),
    )(x, W_enc, b_enc2)


def _mlp_body(h_ref, a_ref, s_ref, w1_ref, b1_ref, w2_ref, b2_ref, o_ref):
    h = jnp.concatenate([h_ref[0], h_ref[1]], axis=1)
    agg = jnp.concatenate([a_ref[0], a_ref[1]], axis=1)
    z = s_ref[0, 0] * h + agg
    y = jnp.maximum(jnp.dot(z, w1_ref[...], preferred_element_type=_f32) + b1_ref[...], 0.0)
    o = jnp.maximum(jnp.dot(y, w2_ref[...], preferred_element_type=_f32) + b2_ref[...], 0.0)
    o_ref[0] = o[:, :128]
    o_ref[1] = o[:, 128:]


def _mlp(h_flat, agg_flat, smul, W1l, b1l, W2l, b2l):
    return pl.pallas_call(
        _mlp_body,
        grid=(NBLK,),
        in_specs=[
            pl.BlockSpec((2, RB, 128), lambda i: (0, i, 0)),
            pl.BlockSpec((2, RB, 128), lambda i: (0, i, 0)),
            pl.BlockSpec((1, 1), lambda i: (0, 0)),
            pl.BlockSpec((HH, 2 * HH), lambda i: (0, 0)),
            pl.BlockSpec((1, 2 * HH), lambda i: (0, 0)),
            pl.BlockSpec((2 * HH, HH), lambda i: (0, 0)),
            pl.BlockSpec((1, HH), lambda i: (0, 0)),
        ],
        out_specs=pl.BlockSpec((2, RB, 128), lambda i: (0, i, 0)),
        out_shape=jax.ShapeDtypeStruct((2, NP, 128), _f32),
    )(h_flat, agg_flat, smul, W1l, b1l, W2l, b2l)


def _pool_body(h0, h1, h2, h3, h4, h5, b_ref, wh1_ref, bh1_ref, wh2_ref,
               bh2_ref, o_ref, pacc, cacc):
    i = pl.program_id(0)

    @pl.when(i == 0)
    def _():
        pacc[...] = jnp.zeros_like(pacc)
        cacc[...] = jnp.zeros_like(cacc)

    b = b_ref[0, 0, :]
    gi = lax.broadcasted_iota(jnp.int32, (GG, RB), 0)
    oneh = (gi == b[None, :]).astype(_f32)
    cnt = jnp.sum(oneh, axis=1, keepdims=True)
    cacc[...] += jnp.broadcast_to(cnt, (GG, 128))
    for l, hr in enumerate([h0, h1, h2, h3, h4, h5]):
        hl = jnp.concatenate([hr[0], hr[1]], axis=1)
        pacc[l] += jnp.dot(oneh, hl, preferred_element_type=_f32)

    @pl.when(i == NBLK - 1)
    def _():
        d = jnp.maximum(cacc[...], 1.0)
        d2 = jnp.concatenate([d, d], axis=1)
        acc = jnp.zeros((GG, HH), _f32)
        for l in range(LL + 1):
            acc += jnp.dot(pacc[l] / d2, wh1_ref[l], preferred_element_type=_f32)
        y = jnp.maximum(acc + bh1_ref[...], 0.0)
        o_ref[...] = jnp.dot(y, wh2_ref[...], preferred_element_type=_f32) + bh2_ref[0, 0]


def _pool_head(hs, batch3, Wh1r, bh12, Wh2p, bh22):
    return pl.pallas_call(
        _pool_body,
        grid=(NBLK,),
        in_specs=[pl.BlockSpec((2, RB, 128), lambda i: (0, i, 0)) for _ in range(6)]
        + [
            pl.BlockSpec((1, 1, RB), lambda i: (i, 0, 0)),
            pl.BlockSpec((LL + 1, HH, HH), lambda i: (0, 0, 0)),
            pl.BlockSpec((1, HH), lambda i: (0, 0)),
            pl.BlockSpec((HH, 128), lambda i: (0, 0)),
            pl.BlockSpec((1, 1), lambda i: (0, 0)),
        ],
        out_specs=pl.BlockSpec((GG, 128), lambda i: (0, 0)),
        out_shape=jax.ShapeDtypeStruct((GG, 128), _f32),
        scratch_shapes=[
            pltpu.VMEM((LL + 1, GG, HH), _f32),
            pltpu.VMEM((GG, 128), _f32),
        ],
    )(*hs, batch3, Wh1r, bh12, Wh2p, bh22)


NSLOT = 4
NPHASE = 4
CPP = CHUNKS_PER_TILE // NPHASE
NGRP = CPP // NSLOT
ACCR = 10112
RPT = ACCR // 16


def _sc_segsum_body(h_hbm, src4_hbm, dst4_hbm, zero_hbm, out_hbm, *scr):
    sidx_all, didx_all = scr[0], scr[1]
    rows = list(scr[2:2 + NSLOT])
    acc = scr[2 + NSLOT]
    gsem = list(scr[3 + NSLOT:3 + 2 * NSLOT])
    ssem = list(scr[3 + 2 * NSLOT:3 + 3 * NSLOT])
    c = lax.axis_index("c")
    s = lax.axis_index("s")
    pltpu.sync_copy(zero_hbm.at[pl.ds(s * RPT, RPT)],
                    acc.at[pl.ds(s * RPT, RPT)])
    plsc.subcore_barrier()

    for p in range(NPHASE):
        pltpu.sync_copy(src4_hbm.at[c, s, p], sidx_all)
        pltpu.sync_copy(dst4_hbm.at[s, p], didx_all)

        for j in range(NSLOT):
            pltpu.async_copy(h_hbm.at[sidx_all.at[j]], rows[j], gsem[j])

        def body(i, carry):
            for j in range(NSLOT):
                k = i * NSLOT + j
                pltpu.make_async_copy(h_hbm.at[sidx_all.at[k]], rows[j], gsem[j]).wait()
                pltpu.async_copy(rows[j], acc.at[didx_all.at[k]], ssem[j], add=True)
            for j in range(NSLOT):
                k = i * NSLOT + j
                pltpu.make_async_copy(rows[j], acc.at[didx_all.at[k]], ssem[j]).wait()

                @pl.when(i < NGRP - 1)
                def _():
                    pltpu.async_copy(h_hbm.at[sidx_all.at[k + NSLOT]], rows[j], gsem[j])
            return carry

        lax.fori_loop(0, NGRP, body, 0)

    plsc.subcore_barrier()
    pltpu.sync_copy(acc.at[pl.ds(s * RPT, RPT)],
                    out_hbm.at[pl.ds(c * NP + s * RPT, RPT)])


def _segsum(h_flat2, src4, dst4, zeros_half):
    mesh = plsc.VectorSubcoreMesh(core_axis_name="c", subcore_axis_name="s")
    f = functools.partial(
        pl.kernel,
        mesh=mesh,
        out_type=jax.ShapeDtypeStruct((FLAT, 128), _f32),
        scratch_types=[
            pltpu.VMEM((CPP, CH), jnp.int32),
            pltpu.VMEM((CPP, CH), jnp.int32),
        ]
        + [pltpu.VMEM((CH, 128), _f32) for _ in range(NSLOT)]
        + [pltpu.VMEM_SHARED((ACCR, 128), _f32)]
        + [pltpu.SemaphoreType.DMA for _ in range(2 * NSLOT)],
    )(_sc_segsum_body)
    return f(h_flat2, src4, dst4, zeros_half)


def kernel(x, edge_index, batch, W_enc, b_enc, W1, b1, W2, b2, eps, gamma,
           beta, rmean, rvar, Wh1, bh1, Wh2, bh2):
    src = edge_index[0]
    dst = edge_index[1]
    pad = EP - EE
    srcp = jnp.concatenate([src, jnp.zeros((pad,), jnp.int32)])
    dstp = jnp.concatenate([dst, jnp.full((pad,), NN, jnp.int32)])
    src4 = jnp.stack([srcp, srcp + NP]).reshape(2, 16, NPHASE, CPP, CH)
    dst4 = dstp.reshape(16, NPHASE, CPP, CH)
    zeros_half = jnp.zeros((NP, 128), _f32)

    scale = gamma / jnp.sqrt(rvar + 1e-5)
    W2f = W2 * scale[:, None, :]
    b2f = (b2 - rmean) * scale + beta

    h_flat = _encode(x, W_enc, b_enc.reshape(1, HH))
    hs = [h_flat]
    for l in range(LL):
        agg = _segsum(h_flat.reshape(FLAT, 128), src4, dst4, zeros_half)
        h_flat = _mlp(h_flat, agg.reshape(2, NP, 128),
                      (1.0 + eps[l]).reshape(1, 1),
                      W1[l], b1[l].reshape(1, 2 * HH),
                      W2f[l], b2f[l].reshape(1, HH))
        hs.append(h_flat)

    batch3 = batch.reshape(NBLK, 1, RB)
    Wh1r = Wh1.reshape(LL + 1, HH, HH)
    Wh2p = jnp.pad(Wh2, ((0, 0), (0, 127)))
    out = _pool_head(hs, batch3, Wh1r, bh1.reshape(1, HH), Wh2p,
                     bh2.reshape(1, 1))
    return out[:, 0]
